# baseline (speedup 1.0000x reference)
import dataclasses
import functools

import jax
import jax.numpy as jnp
from jax import lax
from jax.experimental import pallas as pl
from jax.experimental.pallas import tpu as pltpu
from jax.experimental.pallas import tpu_sc as plsc

E = 4
V = 100000
D = 128
B = 4096
L = 20
NT = E * B * L
NW = 32
PER_W = NT // NW
C = 128
NCH = PER_W // C
LANES = 16

_mesh = plsc.VectorSubcoreMesh(core_axis_name="c", subcore_axis_name="s")

_cp = pltpu.CompilerParams()
if "needs_layout_passes" in pltpu.CompilerParams.__dataclass_fields__:
    _cp = dataclasses.replace(_cp, needs_layout_passes=False)
if "use_tc_tiling_on_sc" in pltpu.CompilerParams.__dataclass_fields__:
    _cp = dataclasses.replace(_cp, use_tc_tiling_on_sc=True)


@functools.partial(
    pl.kernel,
    out_type=jax.ShapeDtypeStruct((NT, D), jnp.float32),
    mesh=_mesh,
    compiler_params=_cp,
    scratch_types=[
        pltpu.VMEM((PER_W,), jnp.int32),
        pltpu.VMEM((PER_W,), jnp.int32),
        pltpu.VMEM((C, D), jnp.float32),
        pltpu.VMEM((C, D), jnp.float32),
        pltpu.VMEM((C,), jnp.float32),
        pltpu.VMEM((C,), jnp.float32),
        pltpu.VMEM((D,), jnp.float32),
        pltpu.SemaphoreType.DMA,
        pltpu.SemaphoreType.DMA,
        pltpu.SemaphoreType.DMA,
        pltpu.SemaphoreType.DMA,
    ],
)
def _sc_embed(idx_hbm, w_hbm, rflat_hbm, s_hbm, out_hbm,
              idx_all, idxr_all, rows0, rows1, rv0, rv1, s_v,
              rsem0, rsem1, vsem0, vsem1):
    rows = (rows0, rows1)
    rv = (rv0, rv1)
    rsem = (rsem0, rsem1)
    vsem = (vsem0, vsem1)

    wid = lax.axis_index("s") * 2 + lax.axis_index("c")
    base = wid * PER_W
    e = wid // (NW // E)

    pltpu.sync_copy(idx_hbm.at[pl.ds(base, PER_W)], idx_all)
    pltpu.sync_copy(s_hbm.at[e], s_v)

    eoffv = jnp.full((LANES,), e * V, jnp.int32)

    @plsc.parallel_loop(0, PER_W, step=LANES, unroll=4)
    def _(t):
        sl = pl.ds(t, LANES)
        idxr_all[sl] = idx_all[sl] + eoffv

    s_regs = [s_v[pl.ds(jj * LANES, LANES)] for jj in range(D // LANES)]

    def issue(g, b):
        sl = pl.ds(g * C, C)
        pltpu.async_copy(w_hbm.at[idx_all.at[sl]], rows[b], rsem[b])
        pltpu.async_copy(rflat_hbm.at[idxr_all.at[sl]], rv[b], vsem[b])

    def wait(g, b):
        sl = pl.ds(g * C, C)
        pltpu.make_async_copy(w_hbm.at[idx_all.at[sl]], rows[b], rsem[b]).wait()
        pltpu.make_async_copy(rflat_hbm.at[idxr_all.at[sl]], rv[b], vsem[b]).wait()

    def compute(b):
        rows_b = rows[b]
        rv_b = rv[b]

        @plsc.parallel_loop(0, C, unroll=8)
        def _(i):
            rvb = plsc.load_gather(rv_b, [jnp.full((LANES,), i, jnp.int32)])
            for jj in range(D // LANES):
                sl = pl.ds(jj * LANES, LANES)
                rows_b[i, sl] = rows_b[i, sl] * rvb * s_regs[jj]

    def store(g, b):
        pltpu.sync_copy(rows[b], out_hbm.at[pl.ds(base + g * C, C)])

    issue(0, 0)

    @pl.loop(0, NCH, step=2)
    def _(g):
        issue(g + 1, 1)
        wait(g, 0)
        compute(0)
        store(g, 0)

        @pl.when(g + 2 < NCH)
        def _():
            issue(g + 2, 0)

        wait(g + 1, 1)
        compute(1)
        store(g + 1, 1)


def kernel(indices, weight, r, s):
    idx_flat = indices.transpose(0, 2, 1).reshape(NT)
    r_flat = r.reshape(-1)
    out = _sc_embed(idx_flat, weight, r_flat, s)
    return out.reshape(E, L, B, D).transpose(0, 2, 1, 3)


# speedup vs baseline: 1.0653x; 1.0653x over previous
import dataclasses
import functools

import jax
import jax.numpy as jnp
from jax import lax
from jax.experimental import pallas as pl
from jax.experimental.pallas import tpu as pltpu
from jax.experimental.pallas import tpu_sc as plsc

E = 4
V = 100000
D = 128
B = 4096
L = 20
NT = E * B * L
NW = 32
PER_W = NT // NW
C = 128
NCH = PER_W // C
LANES = 16

_mesh = plsc.VectorSubcoreMesh(core_axis_name="c", subcore_axis_name="s")

_cp = pltpu.CompilerParams()
if "needs_layout_passes" in pltpu.CompilerParams.__dataclass_fields__:
    _cp = dataclasses.replace(_cp, needs_layout_passes=False)
if "use_tc_tiling_on_sc" in pltpu.CompilerParams.__dataclass_fields__:
    _cp = dataclasses.replace(_cp, use_tc_tiling_on_sc=True)


@functools.partial(
    pl.kernel,
    out_type=jax.ShapeDtypeStruct((NT, D), jnp.float32),
    mesh=_mesh,
    compiler_params=_cp,
    scratch_types=[
        pltpu.VMEM((PER_W,), jnp.int32),
        pltpu.VMEM((PER_W,), jnp.int32),
        pltpu.VMEM((C, D), jnp.float32),
        pltpu.VMEM((C, D), jnp.float32),
        pltpu.VMEM((C, D), jnp.float32),
        pltpu.VMEM((C, D), jnp.float32),
        pltpu.VMEM((C,), jnp.float32),
        pltpu.VMEM((C,), jnp.float32),
        pltpu.VMEM((C,), jnp.float32),
        pltpu.VMEM((C,), jnp.float32),
        pltpu.VMEM((D,), jnp.float32),
        pltpu.SemaphoreType.DMA,
        pltpu.SemaphoreType.DMA,
        pltpu.SemaphoreType.DMA,
        pltpu.SemaphoreType.DMA,
        pltpu.SemaphoreType.DMA,
        pltpu.SemaphoreType.DMA,
        pltpu.SemaphoreType.DMA,
        pltpu.SemaphoreType.DMA,
        pltpu.SemaphoreType.DMA,
        pltpu.SemaphoreType.DMA,
        pltpu.SemaphoreType.DMA,
        pltpu.SemaphoreType.DMA,
    ],
)
def _sc_embed(idx_hbm, w_hbm, rflat_hbm, s_hbm, out_hbm,
              idx_all, idxr_all, rows0, rows1, rows2, rows3,
              rv0, rv1, rv2, rv3, s_v,
              rsem0, rsem1, rsem2, rsem3,
              vsem0, vsem1, vsem2, vsem3,
              ssem0, ssem1, ssem2, ssem3):
    rows = (rows0, rows1, rows2, rows3)
    rv = (rv0, rv1, rv2, rv3)
    rsem = (rsem0, rsem1, rsem2, rsem3)
    vsem = (vsem0, vsem1, vsem2, vsem3)
    ssem = (ssem0, ssem1, ssem2, ssem3)

    wid = lax.axis_index("s") * 2 + lax.axis_index("c")
    base = wid * PER_W
    e = wid // (NW // E)

    pltpu.sync_copy(idx_hbm.at[pl.ds(base, PER_W)], idx_all)
    pltpu.sync_copy(s_hbm.at[e], s_v)

    eoffv = jnp.full((LANES,), e * V, jnp.int32)

    @plsc.parallel_loop(0, PER_W, step=LANES, unroll=4)
    def _(t):
        sl = pl.ds(t, LANES)
        idxr_all[sl] = idx_all[sl] + eoffv

    s_regs = [s_v[pl.ds(jj * LANES, LANES)] for jj in range(D // LANES)]

    def issue(g, b):
        sl = pl.ds(g * C, C)
        pltpu.async_copy(w_hbm.at[idx_all.at[sl]], rows[b], rsem[b])
        pltpu.async_copy(rflat_hbm.at[idxr_all.at[sl]], rv[b], vsem[b])

    def wait(g, b):
        sl = pl.ds(g * C, C)
        pltpu.make_async_copy(w_hbm.at[idx_all.at[sl]], rows[b], rsem[b]).wait()
        pltpu.make_async_copy(rflat_hbm.at[idxr_all.at[sl]], rv[b], vsem[b]).wait()

    def compute(b):
        rows_b = rows[b]
        rv_b = rv[b]

        @plsc.parallel_loop(0, C, unroll=8)
        def _(i):
            rvb = plsc.load_gather(rv_b, [jnp.full((LANES,), i, jnp.int32)])
            for jj in range(D // LANES):
                sl = pl.ds(jj * LANES, LANES)
                rows_b[i, sl] = rows_b[i, sl] * rvb * s_regs[jj]

    def store(g, b):
        pltpu.async_copy(rows[b], out_hbm.at[pl.ds(base + g * C, C)], ssem[b])

    def wait_store(g, b):
        pltpu.make_async_copy(
            rows[b], out_hbm.at[pl.ds(base + g * C, C)], ssem[b]).wait()

    issue(0, 0)
    issue(1, 1)
    issue(2, 2)

    @pl.loop(0, NCH, step=4)
    def _(g0):
        for b in range(4):
            g = g0 + b

            @pl.when(g >= 1)
            def _():
                wait_store(g - 1, (b - 1) % 4)

            @pl.when(g + 3 < NCH)
            def _():
                issue(g + 3, (b + 3) % 4)

            wait(g, b)
            compute(b)
            store(g, b)

    wait_store(NCH - 1, 3)


def kernel(indices, weight, r, s):
    idx_flat = indices.transpose(0, 2, 1).reshape(NT)
    r_flat = r.reshape(-1)
    out = _sc_embed(idx_flat, weight, r_flat, s)
    return out.reshape(E, L, B, D).transpose(0, 2, 1, 3)
